# Initial kernel scaffold; baseline (speedup 1.0000x reference)
import jax
import jax.numpy as jnp
from jax.experimental import pallas as pl


def kernel(features, edge_index, W1, b1, W2, b2, Wfc, bfc):
    raise NotImplementedError("write your pallas kernel here")


# speedup vs baseline: 6.0880x; 6.0880x over previous
import functools

import jax
import jax.numpy as jnp
from jax import lax
from jax.experimental import pallas as pl
from jax.experimental.pallas import tpu as pltpu
from jax.experimental.pallas import tpu_sc as plsc

N_NODES = 10000
N_PAD = 10240
N_EDGES = 320000
FEATS = 128
NC = 2
NS = 16
NW = NC * NS

K = 80
CHUNKS = N_EDGES // (NW * K)
DEG_CHUNKS = N_EDGES // (NS * K)
ROWS_PER_TILE = N_PAD // NS
DEG_W = 16

_sc_mesh = plsc.VectorSubcoreMesh(
    core_axis_name="c", subcore_axis_name="s", num_cores=NC, num_subcores=NS)


@functools.partial(
    pl.kernel,
    out_type=jax.ShapeDtypeStruct((NC, N_PAD, FEATS), jnp.float32),
    mesh=_sc_mesh,
    scratch_types=[
        pltpu.VMEM((DEG_CHUNKS, K), jnp.int32),
        pltpu.VMEM((K, FEATS), jnp.float32),
        pltpu.VMEM_SHARED((N_PAD, FEATS), jnp.float32),
    ],
)
def _deg_kernel(idx4_hbm, ones_hbm, zeros_hbm, deg_hbm, idx_v, ones_v, deg_sh):
    cid = lax.axis_index("c")
    sid = lax.axis_index("s")

    pltpu.sync_copy(ones_hbm, ones_v)
    pltpu.sync_copy(zeros_hbm,
                    deg_sh.at[pl.ds(sid * ROWS_PER_TILE, ROWS_PER_TILE)])
    pltpu.sync_copy(idx4_hbm.at[cid, sid], idx_v)
    plsc.subcore_barrier()

    def step(j, carry):
        pltpu.sync_copy(ones_v, deg_sh.at[idx_v.at[j]], add=True)
        return carry

    lax.fori_loop(0, DEG_CHUNKS, step, 0)
    plsc.subcore_barrier()
    pltpu.sync_copy(deg_sh.at[pl.ds(sid * ROWS_PER_TILE, ROWS_PER_TILE)],
                    deg_hbm.at[cid, pl.ds(sid * ROWS_PER_TILE, ROWS_PER_TILE)])


@functools.partial(
    pl.kernel,
    out_type=jax.ShapeDtypeStruct((NC, N_PAD, FEATS), jnp.float32),
    mesh=_sc_mesh,
    scratch_types=[
        pltpu.VMEM((CHUNKS, K), jnp.int32),
        pltpu.VMEM((CHUNKS, K), jnp.int32),
        pltpu.VMEM((K, FEATS), jnp.float32),
        pltpu.VMEM_SHARED((N_PAD, FEATS), jnp.float32),
        pltpu.SemaphoreType.DMA,
    ],
)
def _edge_kernel(y_hbm, srcs_hbm, dsts_hbm, zeros_hbm, z_hbm,
                 src_v, dst_v, rows_v, z_sh, sem):
    cid = lax.axis_index("c")
    sid = lax.axis_index("s")
    wid = cid * NS + sid

    pltpu.sync_copy(zeros_hbm,
                    z_sh.at[pl.ds(sid * ROWS_PER_TILE, ROWS_PER_TILE)])
    pltpu.sync_copy(srcs_hbm.at[wid], src_v)
    pltpu.sync_copy(dsts_hbm.at[wid], dst_v)
    plsc.subcore_barrier()

    def step(j, carry):
        pltpu.async_copy(y_hbm.at[src_v.at[j]], rows_v, sem).wait()
        pltpu.sync_copy(rows_v, z_sh.at[dst_v.at[j]], add=True)
        return carry

    lax.fori_loop(0, CHUNKS, step, 0)
    plsc.subcore_barrier()
    pltpu.sync_copy(z_sh.at[pl.ds(sid * ROWS_PER_TILE, ROWS_PER_TILE)],
                    z_hbm.at[cid, pl.ds(sid * ROWS_PER_TILE, ROWS_PER_TILE)])


_BLK = 1000
_GRID = N_NODES // _BLK


def _norms(degs_blk):
    out_deg = degs_blk[0, :, 0]
    in_deg = degs_blk[1, :, 0]
    ns = lax.rsqrt(jnp.maximum(out_deg, 1.0))
    nd = lax.rsqrt(jnp.maximum(in_deg, 1.0))
    return ns, nd


def _mm1_body(x_ref, w_ref, degs_ref, y_ref):
    p = jnp.dot(x_ref[...], w_ref[...], preferred_element_type=jnp.float32)
    ns, _ = _norms(degs_ref)
    y_ref[...] = p * ns[:, None]


def _layer2_body(z_ref, degs_ref, b1_ref, w2_ref, y_ref):
    z = z_ref[0] + z_ref[1]
    ns, nd = _norms(degs_ref)
    x = jnp.maximum(z * nd[:, None] + b1_ref[...], 0.0)
    p = jnp.dot(x, w2_ref[...], preferred_element_type=jnp.float32)
    y_ref[...] = p * ns[:, None]


def _final_body(z_ref, degs_ref, b2_ref, wfc_ref, bfc_ref, out_ref, acc_ref):
    i = pl.program_id(0)
    z = z_ref[0] + z_ref[1]
    _, nd = _norms(degs_ref)
    x = jnp.maximum(z * nd[:, None] + b2_ref[...], 0.0)
    s = jnp.sum(x, axis=0, keepdims=True)

    @pl.when(i == 0)
    def _():
        acc_ref[...] = s

    @pl.when(i > 0)
    def _():
        acc_ref[...] = acc_ref[...] + s

    @pl.when(i == _GRID - 1)
    def _():
        pooled = acc_ref[...] * (1.0 / N_NODES)
        out_ref[...] = (
            jnp.dot(pooled, wfc_ref[...], preferred_element_type=jnp.float32)
            + bfc_ref[...])


_degs_spec = pl.BlockSpec((NC, _BLK, FEATS), lambda i: (0, i, 0))
_row_spec = pl.BlockSpec((_BLK, FEATS), lambda i: (i, 0))
_z_spec = pl.BlockSpec((NC, _BLK, FEATS), lambda i: (0, i, 0))


def _mm1(features, w1, degs):
    return pl.pallas_call(
        _mm1_body,
        grid=(_GRID,),
        in_specs=[
            _row_spec,
            pl.BlockSpec((FEATS, FEATS), lambda i: (0, 0)),
            _degs_spec,
        ],
        out_specs=_row_spec,
        out_shape=jax.ShapeDtypeStruct((N_NODES, FEATS), jnp.float32),
    )(features, w1, degs)


def _layer2(z, degs, b1, w2):
    return pl.pallas_call(
        _layer2_body,
        grid=(_GRID,),
        in_specs=[
            _z_spec,
            _degs_spec,
            pl.BlockSpec((1, FEATS), lambda i: (0, 0)),
            pl.BlockSpec((FEATS, FEATS), lambda i: (0, 0)),
        ],
        out_specs=_row_spec,
        out_shape=jax.ShapeDtypeStruct((N_NODES, FEATS), jnp.float32),
    )(z, degs, b1, w2)


def _final(z, degs, b2, wfc, bfc):
    ncls = wfc.shape[1]
    return pl.pallas_call(
        _final_body,
        grid=(_GRID,),
        in_specs=[
            _z_spec,
            _degs_spec,
            pl.BlockSpec((1, FEATS), lambda i: (0, 0)),
            pl.BlockSpec((FEATS, ncls), lambda i: (0, 0)),
            pl.BlockSpec((1, ncls), lambda i: (0, 0)),
        ],
        out_specs=pl.BlockSpec((1, ncls), lambda i: (0, 0)),
        out_shape=jax.ShapeDtypeStruct((1, ncls), jnp.float32),
        scratch_shapes=[pltpu.VMEM((1, FEATS), jnp.float32)],
        compiler_params=pltpu.CompilerParams(
            dimension_semantics=("arbitrary",)),
    )(z, degs, b2, wfc, bfc)


def kernel(features, edge_index, W1, b1, W2, b2, Wfc, bfc):
    ei = edge_index.astype(jnp.int32)
    idx4 = ei.reshape(2, NS, DEG_CHUNKS, K)
    srcs = ei[0].reshape(NW, CHUNKS, K)
    dsts = ei[1].reshape(NW, CHUNKS, K)
    zeros = jnp.zeros((ROWS_PER_TILE, FEATS), jnp.float32)

    ones = jnp.ones((K, FEATS), jnp.float32)
    degs = _deg_kernel(idx4, ones, zeros)
    y1 = _mm1(features, W1, degs)
    z1 = _edge_kernel(y1, srcs, dsts, zeros)
    y2 = _layer2(z1, degs, b1.reshape(1, FEATS), W2)
    z2 = _edge_kernel(y2, srcs, dsts, zeros)
    out = _final(z2, degs, b2.reshape(1, FEATS), Wfc, bfc.reshape(1, -1))
    return out


# speedup vs baseline: 11.5403x; 1.8956x over previous
import functools

import jax
import jax.numpy as jnp
from jax import lax
from jax.experimental import pallas as pl
from jax.experimental.pallas import tpu as pltpu
from jax.experimental.pallas import tpu_sc as plsc

N_NODES = 10000
N_PAD = 10240
N_EDGES = 320000
FEATS = 128
NC = 2
NS = 16
NW = NC * NS

K = 80
CHUNKS = N_EDGES // (NW * K)
ROWS_PER_TILE = N_PAD // NS

_sc_mesh = plsc.VectorSubcoreMesh(
    core_axis_name="c", subcore_axis_name="s", num_cores=NC, num_subcores=NS)


DEG_TILE_E = N_EDGES // NS


@functools.partial(
    pl.kernel,
    out_type=jax.ShapeDtypeStruct((NC, N_PAD), jnp.float32),
    mesh=_sc_mesh,
    compiler_params=pltpu.CompilerParams(needs_layout_passes=False),
    scratch_types=[
        pltpu.VMEM((DEG_TILE_E,), jnp.int32),
        pltpu.VMEM((N_PAD,), jnp.float32),
        pltpu.VMEM((NS, N_PAD // NS), jnp.float32),
        pltpu.VMEM((N_PAD // NS,), jnp.float32),
        pltpu.VMEM_SHARED((NS, N_PAD), jnp.float32),
    ],
)
def _deg_kernel(idx3_hbm, zeros_hbm, deg_hbm, idx_v, hist_v, red_v, res_v,
                hist_sh):
    cid = lax.axis_index("c")
    sid = lax.axis_index("s")
    pltpu.sync_copy(zeros_hbm, hist_v)
    pltpu.sync_copy(idx3_hbm.at[cid, sid], idx_v)
    ones16 = jnp.full((16,), 1.0, jnp.float32)

    def step(i, c):
        idx16 = idx_v[pl.ds(i * 16, 16)]
        plsc.addupdate_scatter(hist_v, [idx16], ones16)
        return c

    lax.fori_loop(0, DEG_TILE_E // 16, step, 0)
    pltpu.sync_copy(hist_v, hist_sh.at[sid])
    plsc.subcore_barrier()
    pltpu.sync_copy(hist_sh.at[:, pl.ds(sid * ROWS_PER_TILE, ROWS_PER_TILE)],
                    red_v)

    def red(g, c):
        acc = jnp.zeros((16,), jnp.float32)
        for r in range(NS):
            acc = acc + red_v[r, pl.ds(g * 16, 16)]
        res_v[pl.ds(g * 16, 16)] = acc
        return c

    lax.fori_loop(0, ROWS_PER_TILE // 16, red, 0)
    pltpu.sync_copy(res_v,
                    deg_hbm.at[cid, pl.ds(sid * ROWS_PER_TILE, ROWS_PER_TILE)])


NB = 4
NPAIR = 8
STEP = 8
MAIN = ((CHUNKS - 5) // STEP) * STEP


@functools.partial(
    pl.kernel,
    out_type=jax.ShapeDtypeStruct((NC, N_PAD, FEATS), jnp.float32),
    mesh=_sc_mesh,
    scratch_types=[
        pltpu.VMEM((NPAIR, 2, K), jnp.int32),
        pltpu.VMEM((NB, K, FEATS), jnp.float32),
        pltpu.VMEM_SHARED((N_PAD, FEATS), jnp.float32),
        [pltpu.SemaphoreType.DMA] * NPAIR,
        [pltpu.SemaphoreType.DMA] * NB,
        [pltpu.SemaphoreType.DMA] * NB,
    ],
)
def _edge_kernel(y_hbm, pairs_hbm, zeros_hbm, z_hbm,
                 pring, rows, z_sh, sp, sg, ss):
    cid = lax.axis_index("c")
    sid = lax.axis_index("s")
    wid = cid * NS + sid

    pltpu.sync_copy(zeros_hbm,
                    z_sh.at[pl.ds(sid * ROWS_PER_TILE, ROWS_PER_TILE)])
    plsc.subcore_barrier()

    def pair_dma(j, p):
        return pltpu.async_copy(pairs_hbm.at[wid, j], pring.at[p], sp[p])

    def wait_pair(j, p):
        pltpu.make_async_copy(pairs_hbm.at[wid, j], pring.at[p], sp[p]).wait()

    def gather(j, p, b):
        del j
        return pltpu.async_copy(y_hbm.at[pring.at[p, 0]], rows.at[b], sg[b])

    def wait_gather(j, p, b):
        del j
        pltpu.make_async_copy(y_hbm.at[pring.at[p, 0]], rows.at[b],
                              sg[b]).wait()

    def scatter(j, p, b):
        del j
        return pltpu.async_copy(rows.at[b], z_sh.at[pring.at[p, 1]], ss[b],
                                add=True)

    def wait_scatter_slot(b):
        pltpu.make_async_copy(rows.at[b], z_sh.at[pring.at[0, 1]], ss[b]).wait()

    for j in range(4):
        pair_dma(j, j)
    for j in range(2):
        wait_pair(j, j)
        gather(j, j, j)

    def slot(j, b, p, i, first):
        wait_gather(j, p, b)
        scatter(j, p, b)
        bn = (b + 2) % NB
        if first:
            @pl.when(i > 0)
            def _():
                wait_scatter_slot(bn)
        else:
            wait_scatter_slot(bn)
        pn = (p + 2) % NPAIR
        wait_pair(j + 2, pn)
        gather(j + 2, pn, bn)
        pair_dma(j + 4, (p + 4) % NPAIR)

    def body(i, carry):
        j0 = i * STEP
        for b in range(STEP):
            slot(j0 + b, b % NB, b, i, first=(b < 2))
        return carry

    lax.fori_loop(0, MAIN // STEP, body, 0)

    for j in range(MAIN, CHUNKS):
        b = j % NB
        p = j % NPAIR
        wait_gather(j, p, b)
        scatter(j, p, b)
        wait_scatter_slot((b + 2) % NB)
        if j + 2 < CHUNKS:
            pn = (p + 2) % NPAIR
            wait_pair(j + 2, pn)
            gather(j + 2, pn, (b + 2) % NB)
        if j + 4 < CHUNKS:
            pair_dma(j + 4, (p + 4) % NPAIR)
    wait_scatter_slot((CHUNKS - 1) % NB)
    wait_scatter_slot((CHUNKS - 2) % NB)
    plsc.subcore_barrier()
    pltpu.sync_copy(z_sh.at[pl.ds(sid * ROWS_PER_TILE, ROWS_PER_TILE)],
                    z_hbm.at[cid, pl.ds(sid * ROWS_PER_TILE, ROWS_PER_TILE)])


_BLK = 1024
_GRID = N_PAD // _BLK


def _norms(degs_blk):
    ns = lax.rsqrt(jnp.maximum(degs_blk[0], 1.0))
    nd = lax.rsqrt(jnp.maximum(degs_blk[1], 1.0))
    return ns, nd


def _mm1_body(x_ref, w_ref, degs_ref, y_ref):
    p = jnp.dot(x_ref[...], w_ref[...], preferred_element_type=jnp.float32)
    ns, _ = _norms(degs_ref)
    y_ref[...] = p * ns[:, None]


def _layer2_body(z_ref, degs_ref, b1_ref, w2_ref, y_ref):
    z = z_ref[0] + z_ref[1]
    ns, nd = _norms(degs_ref)
    x = jnp.maximum(z * nd[:, None] + b1_ref[...], 0.0)
    p = jnp.dot(x, w2_ref[...], preferred_element_type=jnp.float32)
    y_ref[...] = p * ns[:, None]


def _final_body(z_ref, degs_ref, b2_ref, wfc_ref, bfc_ref, out_ref, acc_ref):
    i = pl.program_id(0)
    z = z_ref[0] + z_ref[1]
    _, nd = _norms(degs_ref)
    x = jnp.maximum(z * nd[:, None] + b2_ref[...], 0.0)
    rid = i * _BLK + lax.broadcasted_iota(jnp.int32, (_BLK, 1), 0)
    x = jnp.where(rid < N_NODES, x, 0.0)
    s = jnp.sum(x, axis=0, keepdims=True)

    @pl.when(i == 0)
    def _():
        acc_ref[...] = s

    @pl.when(i > 0)
    def _():
        acc_ref[...] = acc_ref[...] + s

    @pl.when(i == _GRID - 1)
    def _():
        pooled = acc_ref[...] * (1.0 / N_NODES)
        out_ref[...] = (
            jnp.dot(pooled, wfc_ref[...], preferred_element_type=jnp.float32)
            + bfc_ref[...])


_degs_spec = pl.BlockSpec((NC, _BLK), lambda i: (0, i))
_row_spec = pl.BlockSpec((_BLK, FEATS), lambda i: (i, 0))
_z_spec = pl.BlockSpec((NC, _BLK, FEATS), lambda i: (0, i, 0))


def _mm1(features, w1, degs):
    return pl.pallas_call(
        _mm1_body,
        grid=(_GRID,),
        in_specs=[
            _row_spec,
            pl.BlockSpec((FEATS, FEATS), lambda i: (0, 0)),
            _degs_spec,
        ],
        out_specs=_row_spec,
        out_shape=jax.ShapeDtypeStruct((N_PAD, FEATS), jnp.float32),
    )(features, w1, degs)


def _layer2(z, degs, b1, w2):
    return pl.pallas_call(
        _layer2_body,
        grid=(_GRID,),
        in_specs=[
            _z_spec,
            _degs_spec,
            pl.BlockSpec((1, FEATS), lambda i: (0, 0)),
            pl.BlockSpec((FEATS, FEATS), lambda i: (0, 0)),
        ],
        out_specs=_row_spec,
        out_shape=jax.ShapeDtypeStruct((N_PAD, FEATS), jnp.float32),
    )(z, degs, b1, w2)


def _final(z, degs, b2, wfc, bfc):
    ncls = wfc.shape[1]
    return pl.pallas_call(
        _final_body,
        grid=(_GRID,),
        in_specs=[
            _z_spec,
            _degs_spec,
            pl.BlockSpec((1, FEATS), lambda i: (0, 0)),
            pl.BlockSpec((FEATS, ncls), lambda i: (0, 0)),
            pl.BlockSpec((1, ncls), lambda i: (0, 0)),
        ],
        out_specs=pl.BlockSpec((1, ncls), lambda i: (0, 0)),
        out_shape=jax.ShapeDtypeStruct((1, ncls), jnp.float32),
        scratch_shapes=[pltpu.VMEM((1, FEATS), jnp.float32)],
        compiler_params=pltpu.CompilerParams(
            dimension_semantics=("arbitrary",)),
    )(z, degs, b2, wfc, bfc)


def kernel(features, edge_index, W1, b1, W2, b2, Wfc, bfc):
    ei = edge_index.astype(jnp.int32)
    idx3 = ei.reshape(2, NS, DEG_TILE_E)
    pairs = jnp.stack([ei[0].reshape(NW, CHUNKS, K),
                       ei[1].reshape(NW, CHUNKS, K)], axis=2)
    zeros = jnp.zeros((ROWS_PER_TILE, FEATS), jnp.float32)

    degs = _deg_kernel(idx3, jnp.zeros((N_PAD,), jnp.float32))
    features_p = jnp.zeros((N_PAD, FEATS), jnp.float32).at[:N_NODES].set(features)
    y1 = _mm1(features_p, W1, degs)
    z1 = _edge_kernel(y1, pairs, zeros)
    y2 = _layer2(z1, degs, b1.reshape(1, FEATS), W2)
    z2 = _edge_kernel(y2, pairs, zeros)
    out = _final(z2, degs, b2.reshape(1, FEATS), Wfc, bfc.reshape(1, -1))
    return out
